# baseline (speedup 1.0000x reference)
import functools

import jax
import jax.numpy as jnp
from jax import lax
from jax.experimental import pallas as pl
from jax.experimental.pallas import tpu as pltpu
from jax.experimental.pallas import tpu_sc as plsc

_B = 16384
_D = 3
_V = 4096
_L = 16

_info = plsc.get_sparse_core_info()
_NC = _info.num_cores
_NS = _info.num_subcores
_NW = _NC * _NS
_BPW = _B // _NW
_VECS = _BPW // _L


def _body(idx_hbm, tab_hbm, out_hbm, idx_v, tab_v, out_v, sem):
    wid = lax.axis_index("s") * _NC + lax.axis_index("c")
    base = wid * _BPW
    cp_idx = pltpu.async_copy(idx_hbm.at[pl.ds(base, _BPW)], idx_v, sem)
    cp_tab = pltpu.async_copy(tab_hbm, tab_v, sem)
    cp_idx.wait()
    cp_tab.wait()
    for k in range(_VECS):
        i16 = idx_v[pl.ds(_L * k, _L)]
        for c in range(_D):
            c16 = jnp.full((_L,), c, jnp.int32)
            g = plsc.load_gather(tab_v, [c16, i16])
            out_v[c, pl.ds(_L * k, _L)] = g
    pltpu.sync_copy(out_v, out_hbm.at[:, pl.ds(base, _BPW)])


_gather_call = functools.partial(
    pl.kernel,
    mesh=plsc.VectorSubcoreMesh(core_axis_name="c", subcore_axis_name="s"),
    out_type=jax.ShapeDtypeStruct((_D, _B), jnp.float32),
    scratch_types=[
        pltpu.VMEM((_BPW,), jnp.int32),
        pltpu.VMEM((_D, _V), jnp.float32),
        pltpu.VMEM((_D, _BPW), jnp.float32),
        pltpu.SemaphoreType.DMA,
    ],
    compiler_params=pltpu.CompilerParams(needs_layout_passes=False),
)(_body)


@jax.jit
def kernel(channel_indices, embedding):
    out_t = _gather_call(channel_indices.astype(jnp.int32), embedding.T)
    return out_t.T


# speedup vs baseline: 1.0092x; 1.0092x over previous
import functools

import jax
import jax.numpy as jnp
from jax import lax
from jax.experimental import pallas as pl
from jax.experimental.pallas import tpu as pltpu
from jax.experimental.pallas import tpu_sc as plsc

_B = 16384
_D = 3
_V = 4096
_L = 16

_info = plsc.get_sparse_core_info()
_NC = _info.num_cores
_NS = _info.num_subcores
_NW = _NC * _NS
_BPW = _B // _NW
_VECS = _BPW // _L


def _body(idx_hbm, tab_hbm, out_hbm, idx_v, tab_v, out_v, sem_i, sem_t, sem_o):
    wid = lax.axis_index("s") * _NC + lax.axis_index("c")
    base = wid * _BPW
    cp_idx = pltpu.async_copy(idx_hbm.at[pl.ds(base, _BPW)], idx_v, sem_i)
    cp_tab = [
        pltpu.async_copy(
            tab_hbm.at[pl.ds(c * _V, _V)], tab_v.at[pl.ds(c * _V, _V)], sem_t
        )
        for c in range(_D)
    ]
    cp_idx.wait()
    cp_out = []
    for c in range(_D):
        cp_tab[c].wait()
        for k in range(_VECS):
            i16 = idx_v[pl.ds(_L * k, _L)]
            g = plsc.load_gather(tab_v, [i16 + (c * _V) if c else i16])
            out_v[c, pl.ds(_L * k, _L)] = g
        cp_out.append(
            pltpu.async_copy(
                out_v.at[pl.ds(c, 1), :],
                out_hbm.at[pl.ds(c, 1), pl.ds(base, _BPW)],
                sem_o,
            )
        )
    for cp in cp_out:
        cp.wait()


_gather_call = functools.partial(
    pl.kernel,
    mesh=plsc.VectorSubcoreMesh(core_axis_name="c", subcore_axis_name="s"),
    out_type=jax.ShapeDtypeStruct((_D, _B), jnp.float32),
    scratch_types=[
        pltpu.VMEM((_BPW,), jnp.int32),
        pltpu.VMEM((_D * _V,), jnp.float32),
        pltpu.VMEM((_D, _BPW), jnp.float32),
        pltpu.SemaphoreType.DMA,
        pltpu.SemaphoreType.DMA,
        pltpu.SemaphoreType.DMA,
    ],
    compiler_params=pltpu.CompilerParams(needs_layout_passes=False),
)(_body)


@jax.jit
def kernel(channel_indices, embedding):
    tab_t = embedding.T.reshape(-1)
    out_t = _gather_call(channel_indices.astype(jnp.int32), tab_t)
    return out_t.T


# speedup vs baseline: 1.0770x; 1.0671x over previous
import functools

import jax
import jax.numpy as jnp
from jax import lax
from jax.experimental import pallas as pl
from jax.experimental.pallas import tpu as pltpu
from jax.experimental.pallas import tpu_sc as plsc

_B = 16384
_D = 3
_V = 4096
_L = 16

_info = plsc.get_sparse_core_info()
_NC = 1
_NS = _info.num_subcores
_NW = _NC * _NS
_BPW = _B // _NW
_VECS = _BPW // _L


def _body(idx_hbm, tab_hbm, out_hbm, idx_v, tab_v, out_v, sem_i, sem_t, sem_o):
    wid = lax.axis_index("s") * _NC + lax.axis_index("c")
    base = wid * _BPW
    cp_idx = pltpu.async_copy(idx_hbm.at[pl.ds(base, _BPW)], idx_v, sem_i)
    cp_tab = [
        pltpu.async_copy(
            tab_hbm.at[pl.ds(c * _V, _V)], tab_v.at[pl.ds(c * _V, _V)], sem_t
        )
        for c in range(_D)
    ]
    cp_idx.wait()
    cp_out = []
    for c in range(_D):
        cp_tab[c].wait()
        for k in range(_VECS):
            i16 = idx_v[pl.ds(_L * k, _L)]
            g = plsc.load_gather(tab_v, [i16 + (c * _V) if c else i16])
            out_v[c, pl.ds(_L * k, _L)] = g
        cp_out.append(
            pltpu.async_copy(
                out_v.at[pl.ds(c, 1), :],
                out_hbm.at[pl.ds(c, 1), pl.ds(base, _BPW)],
                sem_o,
            )
        )
    for cp in cp_out:
        cp.wait()


_gather_call = functools.partial(
    pl.kernel,
    mesh=plsc.VectorSubcoreMesh(
        core_axis_name="c", subcore_axis_name="s", num_cores=_NC
    ),
    out_type=jax.ShapeDtypeStruct((_D, _B), jnp.float32),
    scratch_types=[
        pltpu.VMEM((_BPW,), jnp.int32),
        pltpu.VMEM((_D * _V,), jnp.float32),
        pltpu.VMEM((_D, _BPW), jnp.float32),
        pltpu.SemaphoreType.DMA,
        pltpu.SemaphoreType.DMA,
        pltpu.SemaphoreType.DMA,
    ],
    compiler_params=pltpu.CompilerParams(needs_layout_passes=False),
)(_body)


@jax.jit
def kernel(channel_indices, embedding):
    tab_t = embedding.T.reshape(-1)
    out_t = _gather_call(channel_indices.astype(jnp.int32), tab_t)
    return out_t.T


# speedup vs baseline: 1.1021x; 1.0233x over previous
import functools

import jax
import jax.numpy as jnp
from jax import lax
from jax.experimental import pallas as pl
from jax.experimental.pallas import tpu as pltpu
from jax.experimental.pallas import tpu_sc as plsc

_B = 16384
_D = 3
_V = 4096
_L = 16

_info = plsc.get_sparse_core_info()
_NC = 1
_NS = _info.num_subcores
_NW = _NC * _NS
_BPW = _B // _NW
_VECS = _BPW // _L


def _body(idx_hbm, tab_hbm, out_hbm, idx_v, tab_v, out_v, sem_i, sem_t, sem_o):
    wid = lax.axis_index("s") * _NC + lax.axis_index("c")
    base = wid * _BPW
    cp_idx = pltpu.async_copy(idx_hbm.at[pl.ds(base, _BPW)], idx_v, sem_i)
    cp_tab = [
        pltpu.async_copy(
            tab_hbm.at[pl.ds(c * _V, _V)], tab_v.at[pl.ds(c * _V, _V)], sem_t
        )
        for c in range(_D)
    ]
    cp_idx.wait()
    cp_out = []
    for c in range(_D):
        cp_tab[c].wait()

        def _chunk(k, _, c=c):
            off = pl.multiple_of(k * _L, _L)
            i16 = idx_v[pl.ds(off, _L)]
            g = plsc.load_gather(tab_v, [i16 + (c * _V) if c else i16])
            out_v[c, pl.ds(off, _L)] = g
            return _

        lax.fori_loop(0, _VECS, _chunk, 0, unroll=4)
        cp_out.append(
            pltpu.async_copy(
                out_v.at[pl.ds(c, 1), :],
                out_hbm.at[pl.ds(c, 1), pl.ds(base, _BPW)],
                sem_o,
            )
        )
    for cp in cp_out:
        cp.wait()


_gather_call = functools.partial(
    pl.kernel,
    mesh=plsc.VectorSubcoreMesh(
        core_axis_name="c", subcore_axis_name="s", num_cores=_NC
    ),
    out_type=jax.ShapeDtypeStruct((_D, _B), jnp.float32),
    scratch_types=[
        pltpu.VMEM((_BPW,), jnp.int32),
        pltpu.VMEM((_D * _V,), jnp.float32),
        pltpu.VMEM((_D, _BPW), jnp.float32),
        pltpu.SemaphoreType.DMA,
        pltpu.SemaphoreType.DMA,
        pltpu.SemaphoreType.DMA,
    ],
    compiler_params=pltpu.CompilerParams(needs_layout_passes=False),
)(_body)


@jax.jit
def kernel(channel_indices, embedding):
    tab_t = embedding.T.reshape(-1)
    out_t = _gather_call(channel_indices.astype(jnp.int32), tab_t)
    return out_t.T


# speedup vs baseline: 1.1089x; 1.0062x over previous
import functools

import jax
import jax.numpy as jnp
from jax import lax
from jax.experimental import pallas as pl
from jax.experimental.pallas import tpu as pltpu
from jax.experimental.pallas import tpu_sc as plsc

_B = 16384
_D = 3
_V = 4096
_L = 16

_info = plsc.get_sparse_core_info()
_NC = 1
_NS = _info.num_subcores
_NW = _NC * _NS
_BPW = _B // _NW
_VECS = _BPW // _L


def _body(idx_hbm, tab_hbm, out_hbm, idx_v, tab_v, out_v, sem_i, sem_t):
    wid = lax.axis_index("s") * _NC + lax.axis_index("c")
    base = wid * _BPW
    cp_idx = pltpu.async_copy(idx_hbm.at[pl.ds(base, _BPW)], idx_v, sem_i)
    cp_tab = [
        pltpu.async_copy(
            tab_hbm.at[pl.ds(c * _V, _V)], tab_v.at[pl.ds(c * _V, _V)], sem_t
        )
        for c in range(_D)
    ]
    cp_idx.wait()
    for cp in cp_tab:
        cp.wait()

    def _chunk(k, _):
        off = pl.multiple_of(k * _L, _L)
        i16 = idx_v[pl.ds(off, _L)]
        for c in range(_D):
            g = plsc.load_gather(tab_v, [i16 + (c * _V) if c else i16])
            out_v[c, pl.ds(off, _L)] = g
        return _

    lax.fori_loop(0, _VECS, _chunk, 0, unroll=4)
    pltpu.sync_copy(out_v, out_hbm.at[:, pl.ds(base, _BPW)])


_gather_call = functools.partial(
    pl.kernel,
    mesh=plsc.VectorSubcoreMesh(
        core_axis_name="c", subcore_axis_name="s", num_cores=_NC
    ),
    out_type=jax.ShapeDtypeStruct((_D, _B), jnp.float32),
    scratch_types=[
        pltpu.VMEM((_BPW,), jnp.int32),
        pltpu.VMEM((_D * _V,), jnp.float32),
        pltpu.VMEM((_D, _BPW), jnp.float32),
        pltpu.SemaphoreType.DMA,
        pltpu.SemaphoreType.DMA,
    ],
    compiler_params=pltpu.CompilerParams(needs_layout_passes=False),
)(_body)


@jax.jit
def kernel(channel_indices, embedding):
    tab_t = embedding.T.reshape(-1)
    out_t = _gather_call(channel_indices.astype(jnp.int32), tab_t)
    return out_t.T


# speedup vs baseline: 1.1148x; 1.0053x over previous
import functools

import jax
import jax.numpy as jnp
from jax import lax
from jax.experimental import pallas as pl
from jax.experimental.pallas import tpu as pltpu
from jax.experimental.pallas import tpu_sc as plsc

_B = 16384
_D = 3
_V = 4096
_L = 16

_info = plsc.get_sparse_core_info()
_NC = 1
_NS = _info.num_subcores
_NW = _NC * _NS
_BPW = _B // _NW
_VECS = _BPW // _L


def _body(idx_hbm, tab_hbm, out_hbm, idx_v, tab_v, out_v, sem_i, sem_t):
    wid = lax.axis_index("s") * _NC + lax.axis_index("c")
    base = wid * _BPW
    cp_idx = pltpu.async_copy(idx_hbm.at[pl.ds(base, _BPW)], idx_v, sem_i)
    cp_tab = [
        pltpu.async_copy(
            tab_hbm.at[pl.ds(c * _V, _V)], tab_v.at[pl.ds(c * _V, _V)], sem_t
        )
        for c in range(_D)
    ]
    cp_idx.wait()
    for cp in cp_tab:
        cp.wait()

    def _chunk(k, _):
        off = pl.multiple_of(k * _L, _L)
        i16 = idx_v[pl.ds(off, _L)]
        for c in range(_D):
            g = plsc.load_gather(tab_v, [i16 + (c * _V) if c else i16])
            out_v[c, pl.ds(off, _L)] = g
        return _

    lax.fori_loop(0, _VECS, _chunk, 0, unroll=8)
    pltpu.sync_copy(out_v, out_hbm.at[:, pl.ds(base, _BPW)])


_gather_call = functools.partial(
    pl.kernel,
    mesh=plsc.VectorSubcoreMesh(
        core_axis_name="c", subcore_axis_name="s", num_cores=_NC
    ),
    out_type=jax.ShapeDtypeStruct((_D, _B), jnp.float32),
    scratch_types=[
        pltpu.VMEM((_BPW,), jnp.int32),
        pltpu.VMEM((_D * _V,), jnp.float32),
        pltpu.VMEM((_D, _BPW), jnp.float32),
        pltpu.SemaphoreType.DMA,
        pltpu.SemaphoreType.DMA,
    ],
    compiler_params=pltpu.CompilerParams(needs_layout_passes=False),
)(_body)


@jax.jit
def kernel(channel_indices, embedding):
    tab_t = embedding.T.reshape(-1)
    out_t = _gather_call(channel_indices.astype(jnp.int32), tab_t)
    return out_t.T
